# baseline (speedup 1.0000x reference)
import jax
import jax.numpy as jnp
from jax import lax
from jax.experimental import pallas as pl
from jax.experimental.pallas import tpu as pltpu
from jax.experimental.pallas import tpu_sc as plsc

PAD = 1
NC, NS, L = 2, 16, 16
B, S, D = 4, 4096, 768
TOT = B * S
ROWS = S + 2
RPT = (ROWS - 2) // NS
SUB = 32
NSUB = RPT // SUB
CPB = S // 512
BIG = 2**30


def _iota():
    return lax.iota(jnp.int32, L)


def _body(
    ids_hbm,
    w_hbm,
    out_hbm,
    row_v,
    pos_v,
    cnt_v,
    cntrd_v,
    dest_v,
    padidx_v,
    zero_v,
    stg0,
    stg1,
    pos_sh,
    cnt_sh,
    gs0,
    gs1,
    os0,
    os1,
    zs,
):
    core = lax.axis_index("c")
    sid = lax.axis_index("s")
    bl = sid // CPB
    c = sid % CPB
    gb = core * 2 + bl
    r0 = 2 + sid * RPT
    r1 = r0 + RPT

    bufs = (stg0, stg1)
    gsems = (gs0, gs1)
    osems = (os0, os1)

    def stage(j):
        src = w_hbm.at[pl.ds(r0 + j * SUB, SUB)]
        return pltpu.async_copy(src, bufs[j % 2], gsems[j % 2])

    st0 = stage(0)
    st1 = stage(1)

    pltpu.sync_copy(ids_hbm.at[pl.ds(gb * S, S)], row_v)

    def count_body(j, acc):
        v = row_v[pl.ds(j * L, L)]
        return acc + jnp.where(v == PAD, 0, 1)

    acc = lax.fori_loop(0, c * 32, count_body, jnp.zeros((L,), jnp.int32))
    lo = jnp.sum(acc)

    cbase = c * 512
    carry = lo
    for i in range(32):
        v = row_v[pl.ds(cbase + i * L, L)]
        mi = jnp.where(v == PAD, 0, 1)
        cs = plsc.cumsum(mi)
        pos_v[bl, pl.ds(cbase + i * L, L)] = (carry + cs) * mi + 1
        carry = carry + jnp.sum(mi)
    hi = carry

    pltpu.sync_copy(pos_v.at[bl, pl.ds(cbase, 512)], pos_sh.at[bl, pl.ds(cbase, 512)])
    z16 = jnp.zeros((L,), jnp.int32)
    cnt_v[0, pl.ds(0, L)] = z16 + lo
    cnt_v[1, pl.ds(0, L)] = z16 + hi
    pltpu.sync_copy(cnt_v, cnt_sh.at[sid])

    padcnt = 512 - (hi - lo)

    @pl.when(padcnt > 0)
    def _pad_path():
        padidx_v[pl.ds(0, L)] = z16 + PAD
        pltpu.async_copy(w_hbm.at[padidx_v], zero_v, zs).wait()
        for i in range(32):
            v = row_v[pl.ds(cbase + i * L, L)]
            mv = v == PAD
            npad = jnp.sum(jnp.where(mv, 1, 0))

            @pl.when(npad > 0)
            def _scatter_pads(i=i, mv=mv):
                toks = gb * S + cbase + i * L + _iota()
                firstpad = jnp.min(jnp.where(mv, toks, BIG))
                dest16 = jnp.where(mv, toks, firstpad)
                pltpu.async_copy(zero_v, out_hbm.at[dest16], zs).wait()

    plsc.subcore_barrier()

    pltpu.sync_copy(pos_sh, pos_v)
    pltpu.sync_copy(cnt_sh, cntrd_v)

    neg1 = z16 - 1
    for r in range(2 * NSUB):
        for q in range(SUB // L):
            dest_v[r, pl.ds(q * L, L)] = neg1

    for bl2 in range(2):
        for cc in range(CPB):
            j = bl2 * CPB + cc
            lo_vec = cntrd_v[j, 0]
            hi_vec = cntrd_v[j, 1]
            inter = jnp.where(
                (lo_vec + 2 < r1) & (hi_vec + 1 >= r0) & (_iota() == 0), 1, 0
            )
            s = jnp.sum(inter)

            @pl.when(s > 0)
            def _scan_chunk(bl2=bl2, cc=cc):
                for i in range(32):
                    p = pos_v[bl2, pl.ds(cc * 512 + i * L, L)]
                    k = p - r0
                    m = (p >= r0) & (p < r1)
                    rowi = jnp.where(m, bl2 * NSUB + (k >> 5), 0)
                    coli = jnp.where(m, k & 31, 0)
                    val = (core * 2 + bl2) * S + cc * 512 + i * L + _iota()
                    plsc.store_scatter(dest_v, [rowi, coli], val, mask=m)

    def scat(j, bl2):
        idx = plsc.Indices(dest_v.at[bl2 * NSUB + j], ignored_value=-1)
        return pltpu.async_copy(bufs[j % 2], out_hbm.at[idx], osems[j % 2])

    st = [st0, st1] + [None] * (NSUB - 2)
    sc = {}
    for j in range(NSUB):
        st[j].wait()
        sc[(j, 0)] = scat(j, 0)
        sc[(j, 1)] = scat(j, 1)
        nj = j + 2
        if nj < NSUB:
            sc[(j, 0)].wait()
            sc[(j, 1)].wait()
            st[nj] = stage(nj)
    for j in (NSUB - 2, NSUB - 1):
        sc[(j, 0)].wait()
        sc[(j, 1)].wait()


@jax.jit
def _sc_bcast(ids_flat, weights):
    mesh = plsc.VectorSubcoreMesh(
        core_axis_name="c", subcore_axis_name="s", num_cores=NC, num_subcores=NS
    )
    return pl.kernel(
        _body,
        out_type=jax.ShapeDtypeStruct((TOT, D), jnp.float32),
        mesh=mesh,
        compiler_params=pltpu.CompilerParams(
            needs_layout_passes=False,
            disable_bounds_checks=True,
            disable_semaphore_checks=True,
            use_tc_tiling_on_sc=False,
        ),
        scratch_types=[
            pltpu.VMEM((S,), jnp.int32),
            pltpu.VMEM((2, S), jnp.int32),
            pltpu.VMEM((2, L), jnp.int32),
            pltpu.VMEM((NS, 2, L), jnp.int32),
            pltpu.VMEM((2 * NSUB, SUB), jnp.int32),
            pltpu.VMEM((L,), jnp.int32),
            pltpu.VMEM((L, D), jnp.float32),
            pltpu.VMEM((SUB, D), jnp.float32),
            pltpu.VMEM((SUB, D), jnp.float32),
            pltpu.VMEM_SHARED((2, S), jnp.int32),
            pltpu.VMEM_SHARED((NS, 2, L), jnp.int32),
            pltpu.SemaphoreType.DMA,
            pltpu.SemaphoreType.DMA,
            pltpu.SemaphoreType.DMA,
            pltpu.SemaphoreType.DMA,
            pltpu.SemaphoreType.DMA,
        ],
    )(ids_flat, weights)


def kernel(input_ids, weights):
    assert input_ids.shape == (B, S)
    assert weights.shape[1] == D
    out = _sc_bcast(input_ids.reshape(-1), weights)
    return out.reshape(B, S, D)


# speedup vs baseline: 2.2140x; 2.2140x over previous
import functools

import jax
import jax.numpy as jnp
from jax import lax
from jax.experimental import pallas as pl
from jax.experimental.pallas import tpu as pltpu
from jax.experimental.pallas import tpu_sc as plsc

PAD = 1
NC, NS, L = 2, 16, 16
NW = NC * NS

B, S, D = 4, 4096, 768
TOT = B * S
PW = TOT // NW
WPB = S // PW
CHUNK = 32
NCHUNK = PW // CHUNK
VPC = CHUNK // L


def _body(ids_hbm, w_hbm, out_hbm, row_v, idx_v, g0, g1, g2, g3, gs0, gs1, gs2, gs3, ss0, ss1, ss2, ss3):
    wid = lax.axis_index("s") * NC + lax.axis_index("c")
    b = wid // WPB
    c = wid % WPB

    pltpu.sync_copy(ids_hbm.at[pl.ds(b * S, S)], row_v)

    def count_body(j, acc):
        v = row_v[pl.ds(j * L, L)]
        return acc + jnp.where(v == PAD, 0, 1)

    acc = lax.fori_loop(0, c * (PW // L), count_body, jnp.zeros((L,), jnp.int32))
    carry = jnp.sum(acc)

    obase = wid * PW
    bufs = (g0, g1, g2, g3)
    gsems = (gs0, gs1, gs2, gs3)
    ssems = (ss0, ss1, ss2, ss3)

    def gather(ci):
        return pltpu.async_copy(w_hbm.at[idx_v.at[ci]], bufs[ci % 4], gsems[ci % 4])

    def scatter(ci):
        dst = out_hbm.at[pl.ds(obase + ci * CHUNK, CHUNK)]
        return pltpu.async_copy(bufs[ci % 4], dst, ssems[ci % 4])

    cbase = c * PW
    gh = [None] * NCHUNK
    sh = [None] * NCHUNK
    for ci in range(NCHUNK):
        for k in range(VPC):
            v = row_v[pl.ds(cbase + (ci * VPC + k) * L, L)]
            mi = jnp.where(v == PAD, 0, 1)
            cs = plsc.cumsum(mi)
            idx_v[ci, pl.ds(k * L, L)] = (carry + cs) * mi + 1
            carry = carry + jnp.sum(mi)
        if ci >= 4:
            sh[ci - 4].wait()
        gh[ci] = gather(ci)
        if ci >= 1:
            gh[ci - 1].wait()
            sh[ci - 1] = scatter(ci - 1)
    gh[NCHUNK - 1].wait()
    sh[NCHUNK - 1] = scatter(NCHUNK - 1)
    for t in range(4):
        sh[NCHUNK - 4 + t].wait()


@jax.jit
def _sc_gather(ids_flat, weights):
    mesh = plsc.VectorSubcoreMesh(
        core_axis_name="c", subcore_axis_name="s", num_cores=NC, num_subcores=NS
    )
    return pl.kernel(
        _body,
        out_type=jax.ShapeDtypeStruct((TOT, D), jnp.float32),
        mesh=mesh,
        compiler_params=pltpu.CompilerParams(
            needs_layout_passes=False,
            disable_bounds_checks=True,
            disable_semaphore_checks=True,
        ),
        scratch_types=[
            pltpu.VMEM((S,), jnp.int32),
            pltpu.VMEM((NCHUNK, CHUNK), jnp.int32),
            pltpu.VMEM((CHUNK, D), jnp.float32),
            pltpu.VMEM((CHUNK, D), jnp.float32),
            pltpu.VMEM((CHUNK, D), jnp.float32),
            pltpu.VMEM((CHUNK, D), jnp.float32),
            pltpu.SemaphoreType.DMA,
            pltpu.SemaphoreType.DMA,
            pltpu.SemaphoreType.DMA,
            pltpu.SemaphoreType.DMA,
            pltpu.SemaphoreType.DMA,
            pltpu.SemaphoreType.DMA,
            pltpu.SemaphoreType.DMA,
            pltpu.SemaphoreType.DMA,
        ],
    )(ids_flat, weights)


def kernel(input_ids, weights):
    assert input_ids.shape == (B, S)
    assert weights.shape[1] == D
    out = _sc_gather(input_ids.reshape(-1), weights)
    return out.reshape(B, S, D)


# speedup vs baseline: 2.2481x; 1.0154x over previous
import functools

import jax
import jax.numpy as jnp
from jax import lax
from jax.experimental import pallas as pl
from jax.experimental.pallas import tpu as pltpu
from jax.experimental.pallas import tpu_sc as plsc

PAD = 1
NC, NS, L = 2, 16, 16
NW = NC * NS

B, S, D = 4, 4096, 768
TOT = B * S
PW = TOT // NW
WPB = S // PW
CHUNK = 32
NCHUNK = PW // CHUNK
VPC = CHUNK // L


def _body(ids_hbm, w_hbm, out_hbm, row_v, idx_v, g0, g1, g2, g3, gs0, gs1, gs2, gs3, ss0, ss1, ss2, ss3):
    wid = lax.axis_index("s") * NC + lax.axis_index("c")
    b = wid // WPB
    c = wid % WPB

    pltpu.sync_copy(ids_hbm.at[pl.ds(b * S, S)], row_v)

    def count_body(j, acc):
        for u in range(4):
            v = row_v[pl.ds(j * 4 * L + u * L, L)]
            acc = acc + jnp.where(v == PAD, 0, 1)
        return acc

    acc = lax.fori_loop(0, c * (PW // (4 * L)), count_body, jnp.zeros((L,), jnp.int32))
    carry = jnp.sum(acc)

    obase = wid * PW
    bufs = (g0, g1, g2, g3)
    gsems = (gs0, gs1, gs2, gs3)
    ssems = (ss0, ss1, ss2, ss3)

    def gather(ci):
        return pltpu.async_copy(w_hbm.at[idx_v.at[ci]], bufs[ci % 4], gsems[ci % 4])

    def scatter(ci):
        dst = out_hbm.at[pl.ds(obase + ci * CHUNK, CHUNK)]
        return pltpu.async_copy(bufs[ci % 4], dst, ssems[ci % 4])

    cbase = c * PW
    gh = [None] * NCHUNK
    sh = [None] * NCHUNK
    for ci in range(NCHUNK):
        for k in range(VPC):
            v = row_v[pl.ds(cbase + (ci * VPC + k) * L, L)]
            mi = jnp.where(v == PAD, 0, 1)
            cs = plsc.cumsum(mi)
            idx_v[ci, pl.ds(k * L, L)] = (carry + cs) * mi + 1
            carry = carry + jnp.sum(mi)
        if ci >= 4:
            sh[ci - 4].wait()
        gh[ci] = gather(ci)
        if ci >= 1:
            gh[ci - 1].wait()
            sh[ci - 1] = scatter(ci - 1)
    gh[NCHUNK - 1].wait()
    sh[NCHUNK - 1] = scatter(NCHUNK - 1)
    for t in range(4):
        sh[NCHUNK - 4 + t].wait()


@jax.jit
def _sc_gather(ids_flat, weights):
    mesh = plsc.VectorSubcoreMesh(
        core_axis_name="c", subcore_axis_name="s", num_cores=NC, num_subcores=NS
    )
    return pl.kernel(
        _body,
        out_type=jax.ShapeDtypeStruct((TOT, D), jnp.float32),
        mesh=mesh,
        compiler_params=pltpu.CompilerParams(
            needs_layout_passes=False,
            disable_bounds_checks=True,
            disable_semaphore_checks=True,
        ),
        scratch_types=[
            pltpu.VMEM((S,), jnp.int32),
            pltpu.VMEM((NCHUNK, CHUNK), jnp.int32),
            pltpu.VMEM((CHUNK, D), jnp.float32),
            pltpu.VMEM((CHUNK, D), jnp.float32),
            pltpu.VMEM((CHUNK, D), jnp.float32),
            pltpu.VMEM((CHUNK, D), jnp.float32),
            pltpu.SemaphoreType.DMA,
            pltpu.SemaphoreType.DMA,
            pltpu.SemaphoreType.DMA,
            pltpu.SemaphoreType.DMA,
            pltpu.SemaphoreType.DMA,
            pltpu.SemaphoreType.DMA,
            pltpu.SemaphoreType.DMA,
            pltpu.SemaphoreType.DMA,
        ],
    )(ids_flat, weights)


def kernel(input_ids, weights):
    assert input_ids.shape == (B, S)
    assert weights.shape[1] == D
    out = _sc_gather(input_ids.reshape(-1), weights)
    return out.reshape(B, S, D)


# speedup vs baseline: 2.2494x; 1.0006x over previous
import jax
import jax.numpy as jnp
from jax import lax
from jax.experimental import pallas as pl
from jax.experimental.pallas import tpu as pltpu
from jax.experimental.pallas import tpu_sc as plsc

PAD = 1
NC, NS, L = 2, 16, 16
NW = NC * NS

B, S, D = 4, 4096, 768
TOT = B * S
PW = TOT // NW
WPB = S // PW
CHUNK = 32
NCHUNK = PW // CHUNK
VPC = CHUNK // L


def _body(ids_hbm, w_hbm, out_hbm, row_v, idx_v, g0, g1, g2, g3, gs0, gs1, gs2, gs3, ss0, ss1, ss2, ss3):
    wid = lax.axis_index("s") * NC + lax.axis_index("c")
    b = wid // WPB
    c = wid % WPB

    pltpu.sync_copy(ids_hbm.at[pl.ds(b * S, S)], row_v)

    def count_body(j, acc):
        for u in range(4):
            v = row_v[pl.ds(j * 4 * L + u * L, L)]
            acc = acc + jnp.where(v == PAD, 0, 1)
        return acc

    acc = lax.fori_loop(0, c * (PW // (4 * L)), count_body, jnp.zeros((L,), jnp.int32))
    carry = jnp.sum(acc)

    obase = wid * PW
    bufs = (g0, g1, g2, g3)
    gsems = (gs0, gs1, gs2, gs3)
    ssems = (ss0, ss1, ss2, ss3)

    def gather(ci):
        return pltpu.async_copy(w_hbm.at[idx_v.at[ci]], bufs[ci % 4], gsems[ci % 4])

    def scatter(ci):
        dst = out_hbm.at[pl.ds(obase + ci * CHUNK, CHUNK)]
        return pltpu.async_copy(bufs[ci % 4], dst, ssems[ci % 4])

    cbase = c * PW
    gh = [None] * NCHUNK
    sh = [None] * NCHUNK
    for ci in range(NCHUNK):
        for k in range(VPC):
            v = row_v[pl.ds(cbase + (ci * VPC + k) * L, L)]
            mi = jnp.where(v == PAD, 0, 1)
            cs = plsc.cumsum(mi)
            idx_v[ci, pl.ds(k * L, L)] = (carry + cs) * mi + 1
            carry = carry + jnp.sum(mi)
        if ci >= 4:
            sh[ci - 4].wait()
        gh[ci] = gather(ci)
        if ci >= 1:
            gh[ci - 1].wait()
            sh[ci - 1] = scatter(ci - 1)
    gh[NCHUNK - 1].wait()
    sh[NCHUNK - 1] = scatter(NCHUNK - 1)
    for t in range(4):
        sh[NCHUNK - 4 + t].wait()


@jax.jit
def _sc_gather(ids_flat, weights):
    mesh = plsc.VectorSubcoreMesh(
        core_axis_name="c", subcore_axis_name="s", num_cores=NC, num_subcores=NS
    )
    return pl.kernel(
        _body,
        out_type=jax.ShapeDtypeStruct((TOT, D), jnp.float32),
        mesh=mesh,
        compiler_params=pltpu.CompilerParams(
            needs_layout_passes=False,
            disable_bounds_checks=True,
            disable_semaphore_checks=True,
        ),
        scratch_types=[
            pltpu.VMEM((S,), jnp.int32),
            pltpu.VMEM((NCHUNK, CHUNK), jnp.int32),
            pltpu.VMEM((CHUNK, D), jnp.float32),
            pltpu.VMEM((CHUNK, D), jnp.float32),
            pltpu.VMEM((CHUNK, D), jnp.float32),
            pltpu.VMEM((CHUNK, D), jnp.float32),
            pltpu.SemaphoreType.DMA,
            pltpu.SemaphoreType.DMA,
            pltpu.SemaphoreType.DMA,
            pltpu.SemaphoreType.DMA,
            pltpu.SemaphoreType.DMA,
            pltpu.SemaphoreType.DMA,
            pltpu.SemaphoreType.DMA,
            pltpu.SemaphoreType.DMA,
        ],
    )(ids_flat, weights)


def kernel(input_ids, weights):
    assert input_ids.shape == (B, S)
    assert weights.shape[1] == D
    out = _sc_gather(input_ids.reshape(-1), weights)
    return out.reshape(B, S, D)
